# Initial kernel scaffold; baseline (speedup 1.0000x reference)
import functools

import jax
import jax.numpy as jnp
from jax import lax
from jax.experimental import pallas as pl
from jax.experimental.pallas import tpu as pltpu
from jax.experimental.pallas import tpu_sc as plsc

B = 16384
D = 64
K = 20
NC = 2
NS = 16
NW = NC * NS
BPW = B // NW
C = 32
NCHUNK = BPW // C


def _sc_scores_kernel(tidx_hbm, cidx_hbm, nidx_hbm, ttab_hbm, ctab_hbm,
                      out_hbm, tidx_v, cidx_v, nidx_v, trows, crows, nrows,
                      scores, sem_t, sem_c, sem_n):
    wid = lax.axis_index("s") * NC + lax.axis_index("c")
    base = wid * BPW

    def chunk_body(ci, carry):
        off = base + ci * C
        pltpu.sync_copy(tidx_hbm.at[pl.ds(off, C)], tidx_v)
        pltpu.sync_copy(cidx_hbm.at[pl.ds(off, C)], cidx_v)
        pltpu.sync_copy(nidx_hbm.at[pl.ds(off * K, C * K)], nidx_v)
        cp_t = pltpu.async_copy(ttab_hbm.at[tidx_v], trows, sem_t)
        cp_c = pltpu.async_copy(ctab_hbm.at[cidx_v], crows, sem_c)
        cp_n = pltpu.async_copy(ctab_hbm.at[nidx_v], nrows, sem_n)
        cp_t.wait()
        cp_c.wait()
        cp_n.wait()

        def elem_body(i, carry2):
            t = [trows[i, pl.ds(16 * j, 16)] for j in range(4)]
            c = [crows[i, pl.ds(16 * j, 16)] for j in range(4)]
            acc = t[0] * c[0] + t[1] * c[1] + t[2] * c[2] + t[3] * c[3]
            scores[i, 0] = jnp.sum(acc)
            for k in range(K):
                r = i * K + k
                acc = t[0] * nrows[r, pl.ds(0, 16)]
                acc += t[1] * nrows[r, pl.ds(16, 16)]
                acc += t[2] * nrows[r, pl.ds(32, 16)]
                acc += t[3] * nrows[r, pl.ds(48, 16)]
                scores[i, 1 + k] = jnp.sum(acc)
            return carry2

        lax.fori_loop(0, C, elem_body, 0)
        pltpu.sync_copy(scores, out_hbm.at[pl.ds(off, C)])
        return carry

    lax.fori_loop(0, NCHUNK, chunk_body, 0)


_sc_scores = functools.partial(
    pl.kernel,
    mesh=plsc.VectorSubcoreMesh(core_axis_name="c", subcore_axis_name="s"),
    out_type=jax.ShapeDtypeStruct((B, K + 1), jnp.float32),
    scratch_types=[
        pltpu.VMEM((C,), jnp.int32),
        pltpu.VMEM((C,), jnp.int32),
        pltpu.VMEM((C * K,), jnp.int32),
        pltpu.VMEM((C, D), jnp.float32),
        pltpu.VMEM((C, D), jnp.float32),
        pltpu.VMEM((C * K, D), jnp.float32),
        pltpu.VMEM((C, K + 1), jnp.float32),
        pltpu.SemaphoreType.DMA,
        pltpu.SemaphoreType.DMA,
        pltpu.SemaphoreType.DMA,
    ],
)(_sc_scores_kernel)


def _loss_kernel(scores_ref, out_ref):
    s = scores_ref[...]
    pos = s[:, 0]
    neg = s[:, 1:]
    per_b = -jax.nn.log_sigmoid(pos) - jnp.sum(jax.nn.log_sigmoid(-neg), axis=1)
    out_ref[0, 0] = jnp.sum(per_b) * (1.0 / B)


def kernel(target_idx, context_idx, negative_indices, target_table, context_table):
    scores = _sc_scores(target_idx.astype(jnp.int32),
                        context_idx.astype(jnp.int32),
                        negative_indices.astype(jnp.int32).reshape(-1),
                        target_table, context_table)
    loss = pl.pallas_call(
        _loss_kernel,
        out_shape=jax.ShapeDtypeStruct((1, 1), jnp.float32),
        in_specs=[pl.BlockSpec(memory_space=pltpu.VMEM)],
        out_specs=pl.BlockSpec(memory_space=pltpu.SMEM),
    )(scores)
    return loss[0, 0]


# speedup vs baseline: 5.0955x; 5.0955x over previous
import functools

import jax
import jax.numpy as jnp
from jax import lax
from jax.experimental import pallas as pl
from jax.experimental.pallas import tpu as pltpu
from jax.experimental.pallas import tpu_sc as plsc

B = 16384
D = 64
K = 20
NC = 2
NS = 16
NW = NC * NS
BPW = B // NW
C = 32
NCHUNK = BPW // C
G = C // 16
NGRP = C * K // 16


def _sc_scores_kernel(tidx_hbm, cidx_hbm, nidx_hbm, ttab_hbm, ctab_hbm,
                      out_hbm, tidx_v, cidx_v, nidx_v, trows, crows, nrows,
                      scores, sem_t, sem_c, sem_n):
    wid = lax.axis_index("s") * NC + lax.axis_index("c")
    base = wid * BPW
    iota = lax.iota(jnp.int32, 16)

    def chunk_body(ci, carry):
        off = base + ci * C
        pltpu.sync_copy(tidx_hbm.at[pl.ds(off, C)], tidx_v)
        pltpu.sync_copy(cidx_hbm.at[pl.ds(off, C)], cidx_v)
        pltpu.sync_copy(nidx_hbm.at[pl.ds(off * K, C * K)], nidx_v)

        for g in range(G):
            tvec = tidx_v[pl.ds(g * 16, 16)]
            cvec = cidx_v[pl.ds(g * 16, 16)]
            for j in range(16):
                pltpu.async_copy(ttab_hbm.at[tvec[j]], trows.at[g * 16 + j],
                                 sem_t)
                pltpu.async_copy(ctab_hbm.at[cvec[j]], crows.at[g * 16 + j],
                                 sem_c)

        def neg_issue(g, carry2):
            nvec = nidx_v[pl.ds(g * 16, 16)]
            for j in range(16):
                pltpu.async_copy(ctab_hbm.at[nvec[j]], nrows.at[g * 16 + j],
                                 sem_n)
            return carry2

        lax.fori_loop(0, NGRP, neg_issue, 0)

        pltpu.make_async_copy(ttab_hbm.at[pl.ds(0, C)], trows, sem_t).wait()
        pltpu.make_async_copy(ctab_hbm.at[pl.ds(0, C)], crows, sem_c).wait()
        pltpu.make_async_copy(ctab_hbm.at[pl.ds(0, C * K)], nrows, sem_n).wait()

        for g in range(G):
            rows = g * 16 + iota
            nrow0 = rows * K
            zero = jnp.zeros((16,), jnp.float32)

            def dbody(d, accs):
                cold = jnp.full((16,), 0, jnp.int32) + d
                tcol = plsc.load_gather(trows, [rows, cold])
                ccol = plsc.load_gather(crows, [rows, cold])
                new = [accs[0] + tcol * ccol]
                for k in range(K):
                    ncol = plsc.load_gather(nrows, [nrow0 + k, cold])
                    new.append(accs[k + 1] + tcol * ncol)
                return tuple(new)

            accs = lax.fori_loop(0, D, dbody, (zero,) * (K + 1))
            for k in range(K + 1):
                plsc.store_scatter(scores, [rows, jnp.full((16,), k, jnp.int32)],
                                   accs[k])

        pltpu.sync_copy(scores, out_hbm.at[pl.ds(off, C)])
        return carry

    lax.fori_loop(0, NCHUNK, chunk_body, 0)


_sc_scores = functools.partial(
    pl.kernel,
    mesh=plsc.VectorSubcoreMesh(core_axis_name="c", subcore_axis_name="s"),
    compiler_params=pltpu.CompilerParams(needs_layout_passes=False),
    out_type=jax.ShapeDtypeStruct((B, K + 1), jnp.float32),
    scratch_types=[
        pltpu.VMEM((C,), jnp.int32),
        pltpu.VMEM((C,), jnp.int32),
        pltpu.VMEM((C * K,), jnp.int32),
        pltpu.VMEM((C, D), jnp.float32),
        pltpu.VMEM((C, D), jnp.float32),
        pltpu.VMEM((C * K, D), jnp.float32),
        pltpu.VMEM((C, K + 1), jnp.float32),
        pltpu.SemaphoreType.DMA,
        pltpu.SemaphoreType.DMA,
        pltpu.SemaphoreType.DMA,
    ],
)(_sc_scores_kernel)


def _loss_kernel(scores_ref, out_ref):
    s = scores_ref[...]
    pos = s[:, 0]
    neg = s[:, 1:]
    per_b = -jax.nn.log_sigmoid(pos) - jnp.sum(jax.nn.log_sigmoid(-neg), axis=1)
    out_ref[0, 0] = jnp.sum(per_b) * (1.0 / B)


def kernel(target_idx, context_idx, negative_indices, target_table, context_table):
    scores = _sc_scores(target_idx.astype(jnp.int32),
                        context_idx.astype(jnp.int32),
                        negative_indices.astype(jnp.int32).reshape(-1),
                        target_table, context_table)
    loss = pl.pallas_call(
        _loss_kernel,
        out_shape=jax.ShapeDtypeStruct((1, 1), jnp.float32),
        in_specs=[pl.BlockSpec(memory_space=pltpu.VMEM)],
        out_specs=pl.BlockSpec(memory_space=pltpu.SMEM),
    )(scores)
    return loss[0, 0]
